# baseline (speedup 1.0000x reference)
import jax
import jax.numpy as jnp
from jax import lax
from jax.experimental import pallas as pl
from jax.experimental.pallas import tpu as pltpu
from jax.experimental.pallas import tpu_sc as plsc

_N = 10000
_E = 320000
_H = 128
_K = 1024

_NC = 2
_NS = 16
_NW = _NC * _NS
_EW = _E // _NW
_CH = 128
_NCH = _EW // _CH
_TL = _EW - _NCH * _CH
_NP = 10240
_TN = _NP // _NS

_mesh = plsc.VectorSubcoreMesh(core_axis_name="c", subcore_axis_name="s",
                               num_cores=_NC, num_subcores=_NS)


def _zero_vec16():
    return jnp.zeros((16,), jnp.float32)


def _deg_body(src_hbm, dst_hbm, out_cnt, sidxs, didxs, sidxT, didxT,
              ones_s, ones_d, acc, gsems, ssems):
    c = lax.axis_index("c")
    s = lax.axis_index("s")
    wid = s * _NC + c
    base_n = s * _TN
    base_e = wid * _EW

    lane = lax.iota(jnp.int32, 16)
    hot0 = jnp.where(lane == 0, 1.0, 0.0)
    hot1 = jnp.where(lane == 1, 1.0, 0.0)
    z16 = _zero_vec16()

    for r in range(_CH):
        for j in range(8):
            ones_s[r, pl.ds(j * 16, 16)] = z16
    for k in range(5):
        pltpu.async_copy(ones_s, acc.at[pl.ds(base_n + k * _CH, _CH)], gsems[0])
    for k in range(5):
        pltpu.make_async_copy(out_cnt.at[0, pl.ds(0, _CH)], ones_s, gsems[0]).wait()

    for r in range(_CH):
        ones_s[r, pl.ds(0, 16)] = hot0
        ones_d[r, pl.ds(0, 16)] = hot1
        for j in range(1, 8):
            ones_d[r, pl.ds(j * 16, 16)] = z16
    plsc.subcore_barrier()

    def prefetch(j, b):
        pltpu.async_copy(src_hbm.at[pl.ds(base_e + j * _CH, _CH)], sidxs[b], gsems[b])
        pltpu.async_copy(dst_hbm.at[pl.ds(base_e + j * _CH, _CH)], didxs[b], gsems[b])

    def wait_idx(b):
        pltpu.make_async_copy(src_hbm.at[pl.ds(0, _CH)], sidxs[b], gsems[b]).wait()
        pltpu.make_async_copy(src_hbm.at[pl.ds(0, _CH)], didxs[b], gsems[b]).wait()

    def wait_scatter(b):
        pltpu.make_async_copy(out_cnt.at[0, pl.ds(0, _CH)], ones_s, ssems[b]).wait()
        pltpu.make_async_copy(out_cnt.at[0, pl.ds(0, _CH)], ones_d, ssems[b]).wait()

    def scatter(b):
        pltpu.async_copy(ones_s, acc.at[sidxs[b]], ssems[b], add=True)
        pltpu.async_copy(ones_d, acc.at[didxs[b]], ssems[b], add=True)

    prefetch(0, 0)
    prefetch(1, 1)

    @pl.loop(0, _NCH, step=3)
    def _(i):
        for k in range(3):
            j = i + k
            b = k
            bn = (k + 2) % 3

            @pl.when(j >= 1)
            def _():
                wait_scatter(bn)

            @pl.when(j + 2 < _NCH)
            def _():
                prefetch(j + 2, bn)

            wait_idx(b)
            scatter(b)

    wait_scatter((_NCH - 1) % 3)

    base_t = base_e + _NCH * _CH
    pltpu.sync_copy(src_hbm.at[pl.ds(base_t, _TL)], sidxT)
    pltpu.sync_copy(dst_hbm.at[pl.ds(base_t, _TL)], didxT)
    pltpu.sync_copy(ones_s.at[pl.ds(0, _TL)], acc.at[sidxT], add=True)
    pltpu.sync_copy(ones_d.at[pl.ds(0, _TL)], acc.at[didxT], add=True)

    plsc.subcore_barrier()
    pltpu.sync_copy(acc.at[pl.ds(base_n, _TN)], out_cnt.at[c, pl.ds(base_n, _TN)])


_deg_call = pl.kernel(
    _deg_body,
    out_type=jax.ShapeDtypeStruct((_NC, _NP, _H), jnp.float32),
    mesh=_mesh,
    scratch_types=[
        [pltpu.VMEM((_CH,), jnp.int32)] * 3,
        [pltpu.VMEM((_CH,), jnp.int32)] * 3,
        pltpu.VMEM((_TL,), jnp.int32),
        pltpu.VMEM((_TL,), jnp.int32),
        pltpu.VMEM((_CH, _H), jnp.float32),
        pltpu.VMEM((_CH, _H), jnp.float32),
        pltpu.VMEM_SHARED((_NP, _H), jnp.float32),
        [pltpu.SemaphoreType.DMA] * 3,
        [pltpu.SemaphoreType.DMA] * 3,
    ],
)


def _agg_body(g_hbm, src_hbm, dst_hbm, ewv, out,
              sidxs, didxs, wbufs, rows, sidxT, didxT, wbT, rowsT,
              acc, gsems, rsems, ssems, tsem):
    c = lax.axis_index("c")
    s = lax.axis_index("s")
    wid = s * _NC + c
    base_n = s * _TN
    base_e = wid * _EW

    z16 = _zero_vec16()
    for r in range(_CH):
        for j in range(8):
            rows[0][r, pl.ds(j * 16, 16)] = z16
    for k in range(5):
        pltpu.async_copy(rows[0], acc.at[pl.ds(base_n + k * _CH, _CH)], rsems[0])
    for k in range(5):
        pltpu.make_async_copy(g_hbm.at[pl.ds(0, _CH)], rows[0], rsems[0]).wait()
    plsc.subcore_barrier()

    def prefetch_idx(j, b):
        pltpu.async_copy(src_hbm.at[pl.ds(base_e + j * _CH, _CH)], sidxs[b], gsems[b])
        pltpu.async_copy(dst_hbm.at[pl.ds(base_e + j * _CH, _CH)], didxs[b], gsems[b])
        pltpu.async_copy(ewv.at[wid, pl.ds(j * 16, 16)], wbufs[b], gsems[b])

    def wait_idx(b):
        pltpu.make_async_copy(src_hbm.at[pl.ds(0, _CH)], sidxs[b], gsems[b]).wait()
        pltpu.make_async_copy(src_hbm.at[pl.ds(0, _CH)], didxs[b], gsems[b]).wait()
        pltpu.make_async_copy(ewv.at[0, pl.ds(0, 16)], wbufs[b], gsems[b]).wait()

    def gather(r, b):
        pltpu.async_copy(g_hbm.at[sidxs[b]], rows[r], rsems[r])

    def wait_rows(r):
        pltpu.make_async_copy(g_hbm.at[pl.ds(0, _CH)], rows[r], rsems[r]).wait()

    def scatter(r, b):
        pltpu.async_copy(rows[r], acc.at[didxs[b]], ssems[r], add=True)

    def wait_scatter(r):
        pltpu.make_async_copy(g_hbm.at[pl.ds(0, _CH)], rows[r], ssems[r]).wait()

    def scale(r, b):
        br = rows[r]
        bw = wbufs[b]

        def edge(e, ecarry):
            w = bw[lax.shift_right_logical(e, 3),
                   pl.ds(lax.shift_left(lax.bitwise_and(e, 7), 4), 16)]
            for j in range(8):
                sl = pl.ds(j * 16, 16)
                br[e, sl] = br[e, sl] * w
            return ecarry

        lax.fori_loop(0, _CH, edge, 0, unroll=2)

    prefetch_idx(0, 0)
    prefetch_idx(1, 1)
    wait_idx(0)
    gather(0, 0)

    @pl.loop(0, _NCH, step=6)
    def _(i):
        for k in range(6):
            j = i + k
            r = k & 1
            r1 = (k + 1) & 1
            i0 = k % 3
            i1 = (k + 1) % 3
            i2 = (k + 2) % 3

            wait_rows(r)
            scale(r, i0)
            scatter(r, i0)

            @pl.when(j >= 1)
            def _():
                wait_scatter(r1)

            @pl.when(j + 1 < _NCH)
            def _():
                wait_idx(i1)
                gather(r1, i1)

            @pl.when(j + 2 < _NCH)
            def _():
                prefetch_idx(j + 2, i2)

    wait_scatter((_NCH - 1) & 1)

    base_t = base_e + _NCH * _CH
    pltpu.sync_copy(src_hbm.at[pl.ds(base_t, _TL)], sidxT)
    pltpu.sync_copy(dst_hbm.at[pl.ds(base_t, _TL)], didxT)
    pltpu.sync_copy(ewv.at[wid, pl.ds(_NCH * 16, 2)], wbT)
    pltpu.async_copy(g_hbm.at[sidxT], rowsT, tsem)
    pltpu.make_async_copy(g_hbm.at[pl.ds(0, _TL)], rowsT, tsem).wait()

    def tedge(e, ecarry):
        w = wbT[lax.shift_right_logical(e, 3),
                pl.ds(lax.shift_left(lax.bitwise_and(e, 7), 4), 16)]
        for j in range(8):
            sl = pl.ds(j * 16, 16)
            rowsT[e, sl] = rowsT[e, sl] * w
        return ecarry

    lax.fori_loop(0, _TL, tedge, 0, unroll=2)
    pltpu.sync_copy(rowsT, acc.at[didxT], add=True)

    plsc.subcore_barrier()
    pltpu.sync_copy(acc.at[pl.ds(base_n, _TN)], out.at[c, pl.ds(base_n, _TN)])


_agg_call = pl.kernel(
    _agg_body,
    out_type=jax.ShapeDtypeStruct((_NC, _NP, _H), jnp.float32),
    mesh=_mesh,
    scratch_types=[
        [pltpu.VMEM((_CH,), jnp.int32)] * 3,
        [pltpu.VMEM((_CH,), jnp.int32)] * 3,
        [pltpu.VMEM((16, 128), jnp.float32)] * 3,
        [pltpu.VMEM((_CH, _H), jnp.float32)] * 2,
        pltpu.VMEM((_TL,), jnp.int32),
        pltpu.VMEM((_TL,), jnp.int32),
        pltpu.VMEM((2, 128), jnp.float32),
        pltpu.VMEM((_TL, _H), jnp.float32),
        pltpu.VMEM_SHARED((_NP, _H), jnp.float32),
        [pltpu.SemaphoreType.DMA] * 3,
        [pltpu.SemaphoreType.DMA] * 2,
        [pltpu.SemaphoreType.DMA] * 2,
        pltpu.SemaphoreType.DMA,
    ],
)


_NB = 512
_GRID = (_N + _NB - 1) // _NB
_EB = _E // _GRID


def _t0_body(feat, w0, b0, w1, ew, cnt, ew_col, g1, norms, ewb):
    inv_max = 1.0 / jnp.max(ew[...])
    deg_s = jnp.maximum(cnt[0, :, 0] + cnt[1, :, 0], 1.0)
    deg_d = jnp.maximum(cnt[0, :, 1] + cnt[1, :, 1], 1.0)
    no = lax.rsqrt(deg_s) * inv_max
    ni = lax.rsqrt(deg_d)
    h1 = jnp.dot(feat[...], w0[...], preferred_element_type=jnp.float32) + b0[...]
    g1[...] = jnp.dot(h1, w1[...], preferred_element_type=jnp.float32) * no[:, None]
    norms[...] = jnp.concatenate([no[:, None], ni[:, None]], axis=1)
    ewb[...] = jnp.broadcast_to(ew_col[...], (_EB, 16))


def _t0(feat, w0, b0, w1, ew2d, cnt, ew_col):
    return pl.pallas_call(
        _t0_body,
        grid=(_GRID,),
        in_specs=[
            pl.BlockSpec((_NB, 7), lambda i: (i, 0)),
            pl.BlockSpec((7, _H), lambda i: (0, 0)),
            pl.BlockSpec((1, _H), lambda i: (0, 0)),
            pl.BlockSpec((_H, _H), lambda i: (0, 0)),
            pl.BlockSpec((_E // 128, 128), lambda i: (0, 0)),
            pl.BlockSpec((_NC, _NB, _H), lambda i: (0, i, 0)),
            pl.BlockSpec((_EB, 1), lambda i: (i, 0)),
        ],
        out_specs=[
            pl.BlockSpec((_NB, _H), lambda i: (i, 0)),
            pl.BlockSpec((_NB, 2), lambda i: (i, 0)),
            pl.BlockSpec((_EB, 16), lambda i: (i, 0)),
        ],
        out_shape=[
            jax.ShapeDtypeStruct((_N, _H), jnp.float32),
            jax.ShapeDtypeStruct((_N, 2), jnp.float32),
            jax.ShapeDtypeStruct((_E, 16), jnp.float32),
        ],
    )(feat, w0, b0, w1, ew2d, cnt, ew_col)


def _t1_body(p, norms, b, w, g):
    ni = norms[:, 1]
    no = norms[:, 0]
    h = jnp.maximum((p[0] + p[1]) * ni[:, None] + b[...], 0.0)
    g[...] = jnp.dot(h, w[...], preferred_element_type=jnp.float32) * no[:, None]


def _t1(p, norms, b, w):
    return pl.pallas_call(
        _t1_body,
        grid=(_GRID,),
        in_specs=[
            pl.BlockSpec((_NC, _NB, _H), lambda i: (0, i, 0)),
            pl.BlockSpec((_NB, 2), lambda i: (i, 0)),
            pl.BlockSpec((1, _H), lambda i: (0, 0)),
            pl.BlockSpec((_H, _H), lambda i: (0, 0)),
        ],
        out_specs=pl.BlockSpec((_NB, _H), lambda i: (i, 0)),
        out_shape=jax.ShapeDtypeStruct((_N, _H), jnp.float32),
    )(p, norms, b, w)


def _t3_body(p, norms, b, cb, h_out, q_out, ind_out, loss):
    i = pl.program_id(0)
    ni = norms[:, 1]
    h = (p[0] + p[1]) * ni[:, None] + b[...]
    cbv = cb[...]
    d2 = (jnp.sum(h * h, axis=1, keepdims=True)
          - 2.0 * lax.dot_general(h, cbv, (((1,), (1,)), ((), ())),
                                  preferred_element_type=jnp.float32)
          + jnp.sum(cbv * cbv, axis=1)[None, :])
    m = jnp.min(d2, axis=1)
    iota = lax.broadcasted_iota(jnp.int32, (_NB, _K), 1)
    ind = jnp.min(jnp.where(d2 == m[:, None], iota, _K), axis=1)
    onehot = (iota == ind[:, None]).astype(jnp.float32)
    q = jnp.dot(onehot, cbv, preferred_element_type=jnp.float32)
    quant = h + (q - h)
    rid = i * _NB + lax.broadcasted_iota(jnp.int32, (_NB, 1), 0)
    valid = rid < _N
    sq = jnp.where(valid, (q - h) ** 2, 0.0)
    part = jnp.sum(sq) * (1.0 / (_N * _H))

    @pl.when(i == 0)
    def _():
        loss[...] = jnp.zeros_like(loss)

    loss[...] += part
    h_out[...] = h
    q_out[...] = quant
    ind_out[...] = ind[:, None]


def _t3(p, norms, b, cb):
    return pl.pallas_call(
        _t3_body,
        grid=(_GRID,),
        in_specs=[
            pl.BlockSpec((_NC, _NB, _H), lambda i: (0, i, 0)),
            pl.BlockSpec((_NB, 2), lambda i: (i, 0)),
            pl.BlockSpec((1, _H), lambda i: (0, 0)),
            pl.BlockSpec((_K, _H), lambda i: (0, 0)),
        ],
        out_specs=[
            pl.BlockSpec((_NB, _H), lambda i: (i, 0)),
            pl.BlockSpec((_NB, _H), lambda i: (i, 0)),
            pl.BlockSpec((_NB, 1), lambda i: (i, 0)),
            pl.BlockSpec((1, 1), lambda i: (0, 0)),
        ],
        out_shape=[
            jax.ShapeDtypeStruct((_N, _H), jnp.float32),
            jax.ShapeDtypeStruct((_N, _H), jnp.float32),
            jax.ShapeDtypeStruct((_N, 1), jnp.int32),
            jax.ShapeDtypeStruct((1, 1), jnp.float32),
        ],
    )(p, norms, b, cb)


def kernel(features, edge_index, edge_weight, epoch, W0, b0, W1, b1, W2, b2, W3, b3, codebook):
    src = edge_index[0]
    dst = edge_index[1]
    cnt = _deg_call(src, dst)
    ew2d = edge_weight.reshape(_E // 128, 128)
    g1, norms, ewb = _t0(features, W0, b0.reshape(1, _H), W1, ew2d, cnt,
                         edge_weight.reshape(_E, 1))
    ewv = ewb.reshape(_NW, _EW * 16 // 128, 128)
    p1 = _agg_call(g1, src, dst, ewv)
    g2 = _t1(p1, norms, b1.reshape(1, _H), W2)
    p2 = _agg_call(g2, src, dst, ewv)
    g3 = _t1(p2, norms, b2.reshape(1, _H), W3)
    p3 = _agg_call(g3, src, dst, ewv)
    h, quant, ind, loss = _t3(p3, norms, b3.reshape(1, _H), codebook)
    return h, quant, jnp.reshape(loss, ()), ind.reshape(_N)


# speedup vs baseline: 1.2427x; 1.2427x over previous
import functools

import jax
import jax.numpy as jnp
from jax import lax
from jax.experimental import pallas as pl
from jax.experimental.pallas import tpu as pltpu
from jax.experimental.pallas import tpu_sc as plsc

_N = 10000
_E = 320000
_H = 128
_K = 1024

_NC = 2
_NS = 16
_NW = _NC * _NS
_EW = _E // _NW
_C = 80
_NCHUNK = _EW // _C
_NP = 10240
_TN = _NP // _NS

_mesh = plsc.VectorSubcoreMesh(core_axis_name="c", subcore_axis_name="s",
                               num_cores=_NC, num_subcores=_NS)


def _zero_vec16():
    return jnp.zeros((16,), jnp.float32)


def _deg_body(src3, dst3, out_cnt, sidxs, didxs, ones_s, ones_d, acc, gsems, ssems):
    c = lax.axis_index("c")
    s = lax.axis_index("s")
    wid = s * _NC + c
    base_n = s * _TN

    lane = lax.iota(jnp.int32, 16)
    hot0 = jnp.where(lane == 0, 1.0, 0.0)
    hot1 = jnp.where(lane == 1, 1.0, 0.0)
    z16 = _zero_vec16()

    for r in range(_C):
        for j in range(8):
            ones_s[r, pl.ds(j * 16, 16)] = z16
    for k in range(8):
        pltpu.async_copy(ones_s, acc.at[pl.ds(base_n + k * _C, _C)], gsems[0])
    for k in range(8):
        pltpu.make_async_copy(out_cnt.at[0, pl.ds(0, _C)], ones_s, gsems[0]).wait()

    for r in range(_C):
        ones_s[r, pl.ds(0, 16)] = hot0
        ones_d[r, pl.ds(0, 16)] = hot1
        for j in range(1, 8):
            ones_d[r, pl.ds(j * 16, 16)] = z16
    plsc.subcore_barrier()

    def prefetch(j, b):
        pltpu.async_copy(src3.at[wid, j], sidxs[b], gsems[b])
        pltpu.async_copy(dst3.at[wid, j], didxs[b], gsems[b])

    def wait_idx(b):
        pltpu.make_async_copy(src3.at[0, 0], sidxs[b], gsems[b]).wait()
        pltpu.make_async_copy(src3.at[0, 0], didxs[b], gsems[b]).wait()

    def wait_scatter(b):
        pltpu.make_async_copy(out_cnt.at[0, pl.ds(0, _C)], ones_s, ssems[b]).wait()
        pltpu.make_async_copy(out_cnt.at[0, pl.ds(0, _C)], ones_d, ssems[b]).wait()

    def scatter(b):
        pltpu.async_copy(ones_s, acc.at[sidxs[b]], ssems[b], add=True)
        pltpu.async_copy(ones_d, acc.at[didxs[b]], ssems[b], add=True)

    prefetch(0, 0)
    prefetch(1, 1)

    @pl.loop(0, _NCHUNK - 4, step=3)
    def _(i):
        for k in range(3):
            j = i + k
            b = k
            bn = (k + 2) % 3

            @pl.when(j >= 1)
            def _():
                wait_scatter(bn)

            prefetch(j + 2, bn)
            wait_idx(b)
            scatter(b)

    for b in (0, 1):
        wait_idx(b)
        scatter(b)
    for b in range(3):
        wait_scatter(b)

    plsc.subcore_barrier()
    pltpu.sync_copy(acc.at[pl.ds(base_n, _TN)], out_cnt.at[c, pl.ds(base_n, _TN)])


_deg_call = pl.kernel(
    _deg_body,
    out_type=jax.ShapeDtypeStruct((_NC, _NP, _H), jnp.float32),
    mesh=_mesh,
    scratch_types=[
        [pltpu.VMEM((_C,), jnp.int32)] * 3,
        [pltpu.VMEM((_C,), jnp.int32)] * 3,
        pltpu.VMEM((_C, _H), jnp.float32),
        pltpu.VMEM((_C, _H), jnp.float32),
        pltpu.VMEM_SHARED((_NP, _H), jnp.float32),
        [pltpu.SemaphoreType.DMA] * 3,
        [pltpu.SemaphoreType.DMA] * 3,
    ],
)


def _agg_body(g_hbm, src_hbm, dst3, ewb4, out,
              sidx_all, didxs, wbufs, rowbufs, acc, gsems, ssems):
    c = lax.axis_index("c")
    s = lax.axis_index("s")
    wid = s * _NC + c
    base_n = s * _TN

    z16 = _zero_vec16()
    for r in range(_C):
        for j in range(8):
            rowbufs[0][r, pl.ds(j * 16, 16)] = z16
    for k in range(8):
        pltpu.async_copy(rowbufs[0], acc.at[pl.ds(base_n + k * _C, _C)], gsems[0])
    for k in range(8):
        pltpu.make_async_copy(g_hbm.at[pl.ds(0, _C)], rowbufs[0], gsems[0]).wait()
    plsc.subcore_barrier()

    pltpu.sync_copy(src_hbm.at[pl.ds(wid * _EW, _EW)], sidx_all)

    def prefetch(j, b):
        pltpu.async_copy(ewb4.at[wid, j], wbufs[b], gsems[b])
        pltpu.async_copy(dst3.at[wid, j], didxs[b], gsems[b])
        pltpu.async_copy(g_hbm.at[sidx_all.at[pl.ds(j * _C, _C)]], rowbufs[b],
                         gsems[b])

    def wait_gather(b):
        pltpu.make_async_copy(ewb4.at[wid, 0], wbufs[b], gsems[b]).wait()
        pltpu.make_async_copy(dst3.at[wid, 0], didxs[b], gsems[b]).wait()
        pltpu.make_async_copy(g_hbm.at[pl.ds(0, _C)], rowbufs[b], gsems[b]).wait()

    def wait_scatter(b):
        pltpu.make_async_copy(g_hbm.at[pl.ds(0, _C)], rowbufs[b], ssems[b]).wait()

    def scale(b):
        br = rowbufs[b]
        bw = wbufs[b]

        def edge(e, ecarry):
            w = bw[lax.shift_right_logical(e, 3),
                   pl.ds(lax.shift_left(lax.bitwise_and(e, 7), 4), 16)]
            for j in range(8):
                sl = pl.ds(j * 16, 16)
                br[e, sl] = br[e, sl] * w
            return ecarry

        lax.fori_loop(0, _C, edge, 0, unroll=2)

    def scatter(b):
        pltpu.async_copy(rowbufs[b], acc.at[didxs[b]], ssems[b], add=True)

    prefetch(0, 0)
    prefetch(1, 1)

    @pl.loop(0, _NCHUNK - 4, step=3)
    def _(i):
        for k in range(3):
            j = i + k
            b = k
            bn = (k + 2) % 3

            @pl.when(j >= 1)
            def _():
                wait_scatter(bn)

            prefetch(j + 2, bn)
            wait_gather(b)
            scale(b)
            scatter(b)

    for b in (0, 1):
        wait_gather(b)
        scale(b)
        scatter(b)
    for b in range(3):
        wait_scatter(b)

    plsc.subcore_barrier()
    pltpu.sync_copy(acc.at[pl.ds(base_n, _TN)], out.at[c, pl.ds(base_n, _TN)])


_agg_call = pl.kernel(
    _agg_body,
    out_type=jax.ShapeDtypeStruct((_NC, _NP, _H), jnp.float32),
    mesh=_mesh,
    scratch_types=[
        pltpu.VMEM((_EW,), jnp.int32),
        [pltpu.VMEM((_C,), jnp.int32)] * 3,
        [pltpu.VMEM((_C * 16 // 128, 128), jnp.float32)] * 3,
        [pltpu.VMEM((_C, _H), jnp.float32)] * 3,
        pltpu.VMEM_SHARED((_NP, _H), jnp.float32),
        [pltpu.SemaphoreType.DMA] * 3,
        [pltpu.SemaphoreType.DMA] * 3,
    ],
)


_NB = 512
_GRID = (_N + _NB - 1) // _NB


_EB = _E // _GRID


def _t0_body(feat, w0, b0, w1, ew, cnt, ew_col, g1, norms, ewb):
    inv_max = 1.0 / jnp.max(ew[...])
    deg_s = jnp.maximum(cnt[0, :, 0] + cnt[1, :, 0], 1.0)
    deg_d = jnp.maximum(cnt[0, :, 1] + cnt[1, :, 1], 1.0)
    no = lax.rsqrt(deg_s) * inv_max
    ni = lax.rsqrt(deg_d)
    h1 = jnp.dot(feat[...], w0[...], preferred_element_type=jnp.float32) + b0[...]
    g1[...] = jnp.dot(h1, w1[...], preferred_element_type=jnp.float32) * no[:, None]
    norms[...] = jnp.concatenate([no[:, None], ni[:, None]], axis=1)
    ewb[...] = jnp.broadcast_to(ew_col[...], (_EB, 16))


def _t0(feat, w0, b0, w1, ew2d, cnt, ew_col):
    return pl.pallas_call(
        _t0_body,
        grid=(_GRID,),
        in_specs=[
            pl.BlockSpec((_NB, 7), lambda i: (i, 0)),
            pl.BlockSpec((7, _H), lambda i: (0, 0)),
            pl.BlockSpec((1, _H), lambda i: (0, 0)),
            pl.BlockSpec((_H, _H), lambda i: (0, 0)),
            pl.BlockSpec((_E // 128, 128), lambda i: (0, 0)),
            pl.BlockSpec((_NC, _NB, _H), lambda i: (0, i, 0)),
            pl.BlockSpec((_EB, 1), lambda i: (i, 0)),
        ],
        out_specs=[
            pl.BlockSpec((_NB, _H), lambda i: (i, 0)),
            pl.BlockSpec((_NB, 2), lambda i: (i, 0)),
            pl.BlockSpec((_EB, 16), lambda i: (i, 0)),
        ],
        out_shape=[
            jax.ShapeDtypeStruct((_N, _H), jnp.float32),
            jax.ShapeDtypeStruct((_N, 2), jnp.float32),
            jax.ShapeDtypeStruct((_E, 16), jnp.float32),
        ],
    )(feat, w0, b0, w1, ew2d, cnt, ew_col)


def _t1_body(p, norms, b, w, g):
    ni = norms[:, 1]
    no = norms[:, 0]
    h = jnp.maximum((p[0] + p[1]) * ni[:, None] + b[...], 0.0)
    g[...] = jnp.dot(h, w[...], preferred_element_type=jnp.float32) * no[:, None]


def _t1(p, norms, b, w):
    return pl.pallas_call(
        _t1_body,
        grid=(_GRID,),
        in_specs=[
            pl.BlockSpec((_NC, _NB, _H), lambda i: (0, i, 0)),
            pl.BlockSpec((_NB, 2), lambda i: (i, 0)),
            pl.BlockSpec((1, _H), lambda i: (0, 0)),
            pl.BlockSpec((_H, _H), lambda i: (0, 0)),
        ],
        out_specs=pl.BlockSpec((_NB, _H), lambda i: (i, 0)),
        out_shape=jax.ShapeDtypeStruct((_N, _H), jnp.float32),
    )(p, norms, b, w)


def _t3_body(p, norms, b, cb, h_out, q_out, ind_out, loss):
    i = pl.program_id(0)
    ni = norms[:, 1]
    h = (p[0] + p[1]) * ni[:, None] + b[...]
    cbv = cb[...]
    d2 = (jnp.sum(h * h, axis=1, keepdims=True)
          - 2.0 * lax.dot_general(h, cbv, (((1,), (1,)), ((), ())),
                                  preferred_element_type=jnp.float32)
          + jnp.sum(cbv * cbv, axis=1)[None, :])
    m = jnp.min(d2, axis=1)
    iota = lax.broadcasted_iota(jnp.int32, (_NB, _K), 1)
    ind = jnp.min(jnp.where(d2 == m[:, None], iota, _K), axis=1)
    onehot = (iota == ind[:, None]).astype(jnp.float32)
    q = jnp.dot(onehot, cbv, preferred_element_type=jnp.float32)
    quant = h + (q - h)
    rid = i * _NB + lax.broadcasted_iota(jnp.int32, (_NB, 1), 0)
    valid = rid < _N
    sq = jnp.where(valid, (q - h) ** 2, 0.0)
    part = jnp.sum(sq) * (1.0 / (_N * _H))

    @pl.when(i == 0)
    def _():
        loss[...] = jnp.zeros_like(loss)

    loss[...] += part
    h_out[...] = h
    q_out[...] = quant
    ind_out[...] = ind[:, None]


def _t3(p, norms, b, cb):
    return pl.pallas_call(
        _t3_body,
        grid=(_GRID,),
        in_specs=[
            pl.BlockSpec((_NC, _NB, _H), lambda i: (0, i, 0)),
            pl.BlockSpec((_NB, 2), lambda i: (i, 0)),
            pl.BlockSpec((1, _H), lambda i: (0, 0)),
            pl.BlockSpec((_K, _H), lambda i: (0, 0)),
        ],
        out_specs=[
            pl.BlockSpec((_NB, _H), lambda i: (i, 0)),
            pl.BlockSpec((_NB, _H), lambda i: (i, 0)),
            pl.BlockSpec((_NB, 1), lambda i: (i, 0)),
            pl.BlockSpec((1, 1), lambda i: (0, 0)),
        ],
        out_shape=[
            jax.ShapeDtypeStruct((_N, _H), jnp.float32),
            jax.ShapeDtypeStruct((_N, _H), jnp.float32),
            jax.ShapeDtypeStruct((_N, 1), jnp.int32),
            jax.ShapeDtypeStruct((1, 1), jnp.float32),
        ],
    )(p, norms, b, cb)


def kernel(features, edge_index, edge_weight, epoch, W0, b0, W1, b1, W2, b2, W3, b3, codebook):
    src = edge_index[0]
    dst = edge_index[1]
    src3 = src.reshape(_NW, _NCHUNK, _C)
    dst3 = dst.reshape(_NW, _NCHUNK, _C)
    cnt = _deg_call(src3, dst3)
    ew2d = edge_weight.reshape(_E // 128, 128)
    g1, norms, ewb = _t0(features, W0, b0.reshape(1, _H), W1, ew2d, cnt,
                         edge_weight.reshape(_E, 1))
    ewb4 = ewb.reshape(_NW, _NCHUNK, _C * 16 // 128, 128)
    p1 = _agg_call(g1, src, dst3, ewb4)
    g2 = _t1(p1, norms, b1.reshape(1, _H), W2)
    p2 = _agg_call(g2, src, dst3, ewb4)
    g3 = _t1(p2, norms, b2.reshape(1, _H), W3)
    p3 = _agg_call(g3, src, dst3, ewb4)
    h, quant, ind, loss = _t3(p3, norms, b3.reshape(1, _H), codebook)
    return h, quant, jnp.reshape(loss, ()), ind.reshape(_N)


# speedup vs baseline: 1.2473x; 1.0036x over previous
import functools

import jax
import jax.numpy as jnp
from jax import lax
from jax.experimental import pallas as pl
from jax.experimental.pallas import tpu as pltpu
from jax.experimental.pallas import tpu_sc as plsc

_N = 10000
_E = 320000
_H = 128
_K = 1024

_NC = 2
_NS = 16
_NW = _NC * _NS
_EW = _E // _NW
_C = 80
_NCHUNK = _EW // _C
_NP = 10240
_TN = _NP // _NS

_mesh = plsc.VectorSubcoreMesh(core_axis_name="c", subcore_axis_name="s",
                               num_cores=_NC, num_subcores=_NS)


def _zero_vec16():
    return jnp.zeros((16,), jnp.float32)


def _deg_body(src3, dst3, out_cnt, sidxs, didxs, ones_s, ones_d, acc, gsems, ssems):
    c = lax.axis_index("c")
    s = lax.axis_index("s")
    wid = s * _NC + c
    base_n = s * _TN

    lane = lax.iota(jnp.int32, 16)
    hot0 = jnp.where(lane == 0, 1.0, 0.0)
    hot1 = jnp.where(lane == 1, 1.0, 0.0)
    z16 = _zero_vec16()

    for r in range(_C):
        for j in range(8):
            ones_s[r, pl.ds(j * 16, 16)] = z16
    for k in range(8):
        pltpu.async_copy(ones_s, acc.at[pl.ds(base_n + k * _C, _C)], ssems[0])

    for r in range(_C):
        ones_d[r, pl.ds(0, 16)] = hot1
        for j in range(1, 8):
            ones_d[r, pl.ds(j * 16, 16)] = z16
    for k in range(8):
        pltpu.make_async_copy(out_cnt.at[0, pl.ds(0, _C)], ones_s, ssems[0]).wait()
    for r in range(_C):
        ones_s[r, pl.ds(0, 16)] = hot0

    def prefetch(j, b):
        pltpu.async_copy(src3.at[wid, j], sidxs[b], gsems[b])
        pltpu.async_copy(dst3.at[wid, j], didxs[b], gsems[b])

    def wait_idx(b):
        pltpu.make_async_copy(src3.at[0, 0], sidxs[b], gsems[b]).wait()
        pltpu.make_async_copy(src3.at[0, 0], didxs[b], gsems[b]).wait()

    def wait_scatter(b):
        pltpu.make_async_copy(out_cnt.at[0, pl.ds(0, _C)], ones_s, ssems[b]).wait()
        pltpu.make_async_copy(out_cnt.at[0, pl.ds(0, _C)], ones_d, ssems[b]).wait()

    def scatter(b):
        pltpu.async_copy(ones_s, acc.at[sidxs[b]], ssems[b], add=True)
        pltpu.async_copy(ones_d, acc.at[didxs[b]], ssems[b], add=True)

    prefetch(0, 0)
    prefetch(1, 1)
    plsc.subcore_barrier()

    @pl.loop(0, _NCHUNK - 4, step=3)
    def _(i):
        for k in range(3):
            j = i + k
            b = k
            bn = (k + 2) % 3

            @pl.when(j >= 1)
            def _():
                wait_scatter(bn)

            prefetch(j + 2, bn)
            wait_idx(b)
            scatter(b)

    for b in (0, 1):
        wait_idx(b)
        scatter(b)
    for b in range(3):
        wait_scatter(b)

    plsc.subcore_barrier()
    pltpu.sync_copy(acc.at[pl.ds(base_n, _TN)], out_cnt.at[c, pl.ds(base_n, _TN)])


_deg_call = pl.kernel(
    _deg_body,
    out_type=jax.ShapeDtypeStruct((_NC, _NP, _H), jnp.float32),
    mesh=_mesh,
    scratch_types=[
        [pltpu.VMEM((_C,), jnp.int32)] * 3,
        [pltpu.VMEM((_C,), jnp.int32)] * 3,
        pltpu.VMEM((_C, _H), jnp.float32),
        pltpu.VMEM((_C, _H), jnp.float32),
        pltpu.VMEM_SHARED((_NP, _H), jnp.float32),
        [pltpu.SemaphoreType.DMA] * 3,
        [pltpu.SemaphoreType.DMA] * 3,
    ],
)


def _agg_body(g_hbm, src_hbm, dst3, ewb4, out,
              sidx_all, didxs, wbufs, rowbufs, acc, gsems, ssems):
    c = lax.axis_index("c")
    s = lax.axis_index("s")
    wid = s * _NC + c
    base_n = s * _TN

    z16 = _zero_vec16()
    for r in range(_C):
        for j in range(8):
            rowbufs[0][r, pl.ds(j * 16, 16)] = z16
    for k in range(8):
        pltpu.async_copy(rowbufs[0], acc.at[pl.ds(base_n + k * _C, _C)], ssems[0])
    pltpu.sync_copy(src_hbm.at[pl.ds(wid * _EW, _EW)], sidx_all)
    for k in range(8):
        pltpu.make_async_copy(g_hbm.at[pl.ds(0, _C)], rowbufs[0], ssems[0]).wait()

    def prefetch(j, b):
        pltpu.async_copy(ewb4.at[wid, j], wbufs[b], gsems[b])
        pltpu.async_copy(dst3.at[wid, j], didxs[b], gsems[b])
        pltpu.async_copy(g_hbm.at[sidx_all.at[pl.ds(j * _C, _C)]], rowbufs[b],
                         gsems[b])

    def wait_gather(b):
        pltpu.make_async_copy(ewb4.at[wid, 0], wbufs[b], gsems[b]).wait()
        pltpu.make_async_copy(dst3.at[wid, 0], didxs[b], gsems[b]).wait()
        pltpu.make_async_copy(g_hbm.at[pl.ds(0, _C)], rowbufs[b], gsems[b]).wait()

    def wait_scatter(b):
        pltpu.make_async_copy(g_hbm.at[pl.ds(0, _C)], rowbufs[b], ssems[b]).wait()

    def scale(b):
        br = rowbufs[b]
        bw = wbufs[b]

        def edge(e, ecarry):
            w = bw[lax.shift_right_logical(e, 3),
                   pl.ds(lax.shift_left(lax.bitwise_and(e, 7), 4), 16)]
            for j in range(8):
                sl = pl.ds(j * 16, 16)
                br[e, sl] = br[e, sl] * w
            return ecarry

        lax.fori_loop(0, _C, edge, 0, unroll=2)

    def scatter(b):
        pltpu.async_copy(rowbufs[b], acc.at[didxs[b]], ssems[b], add=True)

    prefetch(0, 0)
    prefetch(1, 1)
    plsc.subcore_barrier()

    @pl.loop(0, _NCHUNK - 4, step=3)
    def _(i):
        for k in range(3):
            j = i + k
            b = k
            bn = (k + 2) % 3

            @pl.when(j >= 1)
            def _():
                wait_scatter(bn)

            prefetch(j + 2, bn)
            wait_gather(b)
            scale(b)
            scatter(b)

    for b in (0, 1):
        wait_gather(b)
        scale(b)
        scatter(b)
    for b in range(3):
        wait_scatter(b)

    plsc.subcore_barrier()
    pltpu.sync_copy(acc.at[pl.ds(base_n, _TN)], out.at[c, pl.ds(base_n, _TN)])


_agg_call = pl.kernel(
    _agg_body,
    out_type=jax.ShapeDtypeStruct((_NC, _NP, _H), jnp.float32),
    mesh=_mesh,
    scratch_types=[
        pltpu.VMEM((_EW,), jnp.int32),
        [pltpu.VMEM((_C,), jnp.int32)] * 3,
        [pltpu.VMEM((_C * 16 // 128, 128), jnp.float32)] * 3,
        [pltpu.VMEM((_C, _H), jnp.float32)] * 3,
        pltpu.VMEM_SHARED((_NP, _H), jnp.float32),
        [pltpu.SemaphoreType.DMA] * 3,
        [pltpu.SemaphoreType.DMA] * 3,
    ],
)


_NB = 512
_GRID = (_N + _NB - 1) // _NB


_EB = _E // _GRID


def _t0_body(feat, w0, b0, w1, ew, cnt, ew_col, g1, norms, ewb):
    inv_max = 1.0 / jnp.max(ew[...])
    deg_s = jnp.maximum(cnt[0, :, 0] + cnt[1, :, 0], 1.0)
    deg_d = jnp.maximum(cnt[0, :, 1] + cnt[1, :, 1], 1.0)
    no = lax.rsqrt(deg_s) * inv_max
    ni = lax.rsqrt(deg_d)
    h1 = jnp.dot(feat[...], w0[...], preferred_element_type=jnp.float32) + b0[...]
    g1[...] = jnp.dot(h1, w1[...], preferred_element_type=jnp.float32) * no[:, None]
    norms[...] = jnp.concatenate([no[:, None], ni[:, None]], axis=1)
    ewb[...] = jnp.broadcast_to(ew_col[...], (_EB, 16))


def _t0(feat, w0, b0, w1, ew2d, cnt, ew_col):
    return pl.pallas_call(
        _t0_body,
        grid=(_GRID,),
        in_specs=[
            pl.BlockSpec((_NB, 7), lambda i: (i, 0)),
            pl.BlockSpec((7, _H), lambda i: (0, 0)),
            pl.BlockSpec((1, _H), lambda i: (0, 0)),
            pl.BlockSpec((_H, _H), lambda i: (0, 0)),
            pl.BlockSpec((_E // 128, 128), lambda i: (0, 0)),
            pl.BlockSpec((_NC, _NB, _H), lambda i: (0, i, 0)),
            pl.BlockSpec((_EB, 1), lambda i: (i, 0)),
        ],
        out_specs=[
            pl.BlockSpec((_NB, _H), lambda i: (i, 0)),
            pl.BlockSpec((_NB, 2), lambda i: (i, 0)),
            pl.BlockSpec((_EB, 16), lambda i: (i, 0)),
        ],
        out_shape=[
            jax.ShapeDtypeStruct((_N, _H), jnp.float32),
            jax.ShapeDtypeStruct((_N, 2), jnp.float32),
            jax.ShapeDtypeStruct((_E, 16), jnp.float32),
        ],
    )(feat, w0, b0, w1, ew2d, cnt, ew_col)


def _t1_body(p, norms, b, w, g):
    ni = norms[:, 1]
    no = norms[:, 0]
    h = jnp.maximum((p[0] + p[1]) * ni[:, None] + b[...], 0.0)
    g[...] = jnp.dot(h, w[...], preferred_element_type=jnp.float32) * no[:, None]


def _t1(p, norms, b, w):
    return pl.pallas_call(
        _t1_body,
        grid=(_GRID,),
        in_specs=[
            pl.BlockSpec((_NC, _NB, _H), lambda i: (0, i, 0)),
            pl.BlockSpec((_NB, 2), lambda i: (i, 0)),
            pl.BlockSpec((1, _H), lambda i: (0, 0)),
            pl.BlockSpec((_H, _H), lambda i: (0, 0)),
        ],
        out_specs=pl.BlockSpec((_NB, _H), lambda i: (i, 0)),
        out_shape=jax.ShapeDtypeStruct((_N, _H), jnp.float32),
    )(p, norms, b, w)


def _t3_body(p, norms, b, cb, h_out, q_out, ind_out, loss):
    i = pl.program_id(0)
    ni = norms[:, 1]
    h = (p[0] + p[1]) * ni[:, None] + b[...]
    cbv = cb[...]
    d2 = (jnp.sum(h * h, axis=1, keepdims=True)
          - 2.0 * lax.dot_general(h, cbv, (((1,), (1,)), ((), ())),
                                  preferred_element_type=jnp.float32)
          + jnp.sum(cbv * cbv, axis=1)[None, :])
    m = jnp.min(d2, axis=1)
    iota = lax.broadcasted_iota(jnp.int32, (_NB, _K), 1)
    ind = jnp.min(jnp.where(d2 == m[:, None], iota, _K), axis=1)
    onehot = (iota == ind[:, None]).astype(jnp.float32)
    q = jnp.dot(onehot, cbv, preferred_element_type=jnp.float32)
    quant = h + (q - h)
    rid = i * _NB + lax.broadcasted_iota(jnp.int32, (_NB, 1), 0)
    valid = rid < _N
    sq = jnp.where(valid, (q - h) ** 2, 0.0)
    part = jnp.sum(sq) * (1.0 / (_N * _H))

    @pl.when(i == 0)
    def _():
        loss[...] = jnp.zeros_like(loss)

    loss[...] += part
    h_out[...] = h
    q_out[...] = quant
    ind_out[...] = ind[:, None]


def _t3(p, norms, b, cb):
    return pl.pallas_call(
        _t3_body,
        grid=(_GRID,),
        in_specs=[
            pl.BlockSpec((_NC, _NB, _H), lambda i: (0, i, 0)),
            pl.BlockSpec((_NB, 2), lambda i: (i, 0)),
            pl.BlockSpec((1, _H), lambda i: (0, 0)),
            pl.BlockSpec((_K, _H), lambda i: (0, 0)),
        ],
        out_specs=[
            pl.BlockSpec((_NB, _H), lambda i: (i, 0)),
            pl.BlockSpec((_NB, _H), lambda i: (i, 0)),
            pl.BlockSpec((_NB, 1), lambda i: (i, 0)),
            pl.BlockSpec((1, 1), lambda i: (0, 0)),
        ],
        out_shape=[
            jax.ShapeDtypeStruct((_N, _H), jnp.float32),
            jax.ShapeDtypeStruct((_N, _H), jnp.float32),
            jax.ShapeDtypeStruct((_N, 1), jnp.int32),
            jax.ShapeDtypeStruct((1, 1), jnp.float32),
        ],
    )(p, norms, b, cb)


def kernel(features, edge_index, edge_weight, epoch, W0, b0, W1, b1, W2, b2, W3, b3, codebook):
    src = edge_index[0]
    dst = edge_index[1]
    src3 = src.reshape(_NW, _NCHUNK, _C)
    dst3 = dst.reshape(_NW, _NCHUNK, _C)
    cnt = _deg_call(src3, dst3)
    ew2d = edge_weight.reshape(_E // 128, 128)
    g1, norms, ewb = _t0(features, W0, b0.reshape(1, _H), W1, ew2d, cnt,
                         edge_weight.reshape(_E, 1))
    ewb4 = ewb.reshape(_NW, _NCHUNK, _C * 16 // 128, 128)
    p1 = _agg_call(g1, src, dst3, ewb4)
    g2 = _t1(p1, norms, b1.reshape(1, _H), W2)
    p2 = _agg_call(g2, src, dst3, ewb4)
    g3 = _t1(p2, norms, b2.reshape(1, _H), W3)
    p3 = _agg_call(g3, src, dst3, ewb4)
    h, quant, ind, loss = _t3(p3, norms, b3.reshape(1, _H), codebook)
    return h, quant, jnp.reshape(loss, ()), ind.reshape(_N)


# speedup vs baseline: 1.2506x; 1.0027x over previous
import functools

import jax
import jax.numpy as jnp
from jax import lax
from jax.experimental import pallas as pl
from jax.experimental.pallas import tpu as pltpu
from jax.experimental.pallas import tpu_sc as plsc

_N = 10000
_E = 320000
_H = 128
_K = 1024

_NC = 2
_NS = 16
_NW = _NC * _NS
_EW = _E // _NW
_C = 80
_NCHUNK = _EW // _C
_NP = 10240
_TN = _NP // _NS

_mesh = plsc.VectorSubcoreMesh(core_axis_name="c", subcore_axis_name="s",
                               num_cores=_NC, num_subcores=_NS)


def _zero_vec16():
    return jnp.zeros((16,), jnp.float32)


def _deg_body(src3, dst3, out_cnt, sidxs, didxs, ones_s, ones_d, acc, gsems, ssems):
    c = lax.axis_index("c")
    s = lax.axis_index("s")
    wid = s * _NC + c
    base_n = s * _TN

    lane = lax.iota(jnp.int32, 16)
    hot0 = jnp.where(lane == 0, 1.0, 0.0)
    hot1 = jnp.where(lane == 1, 1.0, 0.0)
    z16 = _zero_vec16()

    @pl.loop(0, _C)
    def _(r):
        for j in range(8):
            ones_s[r, pl.ds(j * 16, 16)] = z16

    for k in range(8):
        pltpu.async_copy(ones_s, acc.at[pl.ds(base_n + k * _C, _C)], ssems[0])

    @pl.loop(0, _C)
    def _(r):
        ones_d[r, pl.ds(0, 16)] = hot1
        for j in range(1, 8):
            ones_d[r, pl.ds(j * 16, 16)] = z16

    for k in range(8):
        pltpu.make_async_copy(out_cnt.at[0, pl.ds(0, _C)], ones_s, ssems[0]).wait()

    @pl.loop(0, _C)
    def _(r):
        ones_s[r, pl.ds(0, 16)] = hot0

    def prefetch(j, b):
        pltpu.async_copy(src3.at[wid, j], sidxs[b], gsems[b])
        pltpu.async_copy(dst3.at[wid, j], didxs[b], gsems[b])

    def wait_idx(b):
        pltpu.make_async_copy(src3.at[0, 0], sidxs[b], gsems[b]).wait()
        pltpu.make_async_copy(src3.at[0, 0], didxs[b], gsems[b]).wait()

    def wait_scatter(b):
        pltpu.make_async_copy(out_cnt.at[0, pl.ds(0, _C)], ones_s, ssems[b]).wait()
        pltpu.make_async_copy(out_cnt.at[0, pl.ds(0, _C)], ones_d, ssems[b]).wait()

    def scatter(b):
        pltpu.async_copy(ones_s, acc.at[sidxs[b]], ssems[b], add=True)
        pltpu.async_copy(ones_d, acc.at[didxs[b]], ssems[b], add=True)

    prefetch(0, 0)
    prefetch(1, 1)
    plsc.subcore_barrier()

    @pl.loop(0, _NCHUNK - 4, step=3)
    def _(i):
        for k in range(3):
            j = i + k
            b = k
            bn = (k + 2) % 3

            @pl.when(j >= 1)
            def _():
                wait_scatter(bn)

            prefetch(j + 2, bn)
            wait_idx(b)
            scatter(b)

    for b in (0, 1):
        wait_idx(b)
        scatter(b)
    for b in range(3):
        wait_scatter(b)

    plsc.subcore_barrier()
    pltpu.sync_copy(acc.at[pl.ds(base_n, _TN)], out_cnt.at[c, pl.ds(base_n, _TN)])


_deg_call = pl.kernel(
    _deg_body,
    out_type=jax.ShapeDtypeStruct((_NC, _NP, _H), jnp.float32),
    mesh=_mesh,
    scratch_types=[
        [pltpu.VMEM((_C,), jnp.int32)] * 3,
        [pltpu.VMEM((_C,), jnp.int32)] * 3,
        pltpu.VMEM((_C, _H), jnp.float32),
        pltpu.VMEM((_C, _H), jnp.float32),
        pltpu.VMEM_SHARED((_NP, _H), jnp.float32),
        [pltpu.SemaphoreType.DMA] * 3,
        [pltpu.SemaphoreType.DMA] * 3,
    ],
)


def _agg_body(g_hbm, src_hbm, dst3, ewb4, out,
              sidx_all, didxs, wbufs, rowbufs, acc, gsems, ssems):
    c = lax.axis_index("c")
    s = lax.axis_index("s")
    wid = s * _NC + c
    base_n = s * _TN

    z16 = _zero_vec16()

    @pl.loop(0, _C)
    def _(r):
        for j in range(8):
            rowbufs[0][r, pl.ds(j * 16, 16)] = z16

    for k in range(8):
        pltpu.async_copy(rowbufs[0], acc.at[pl.ds(base_n + k * _C, _C)], ssems[0])
    pltpu.sync_copy(src_hbm.at[pl.ds(wid * _EW, _EW)], sidx_all)
    for k in range(8):
        pltpu.make_async_copy(g_hbm.at[pl.ds(0, _C)], rowbufs[0], ssems[0]).wait()

    def prefetch(j, b):
        pltpu.async_copy(ewb4.at[wid, j], wbufs[b], gsems[b])
        pltpu.async_copy(dst3.at[wid, j], didxs[b], gsems[b])
        pltpu.async_copy(g_hbm.at[sidx_all.at[pl.ds(j * _C, _C)]], rowbufs[b],
                         gsems[b])

    def wait_gather(b):
        pltpu.make_async_copy(ewb4.at[wid, 0], wbufs[b], gsems[b]).wait()
        pltpu.make_async_copy(dst3.at[wid, 0], didxs[b], gsems[b]).wait()
        pltpu.make_async_copy(g_hbm.at[pl.ds(0, _C)], rowbufs[b], gsems[b]).wait()

    def wait_scatter(b):
        pltpu.make_async_copy(g_hbm.at[pl.ds(0, _C)], rowbufs[b], ssems[b]).wait()

    def scale(b):
        br = rowbufs[b]
        bw = wbufs[b]

        def edge(e, ecarry):
            w = bw[lax.shift_right_logical(e, 3),
                   pl.ds(lax.shift_left(lax.bitwise_and(e, 7), 4), 16)]
            for j in range(8):
                sl = pl.ds(j * 16, 16)
                br[e, sl] = br[e, sl] * w
            return ecarry

        lax.fori_loop(0, _C, edge, 0, unroll=2)

    def scatter(b):
        pltpu.async_copy(rowbufs[b], acc.at[didxs[b]], ssems[b], add=True)

    prefetch(0, 0)
    prefetch(1, 1)
    plsc.subcore_barrier()

    @pl.loop(0, _NCHUNK - 4, step=3)
    def _(i):
        for k in range(3):
            j = i + k
            b = k
            bn = (k + 2) % 3

            @pl.when(j >= 1)
            def _():
                wait_scatter(bn)

            prefetch(j + 2, bn)
            wait_gather(b)
            scale(b)
            scatter(b)

    for b in (0, 1):
        wait_gather(b)
        scale(b)
        scatter(b)
    for b in range(3):
        wait_scatter(b)

    plsc.subcore_barrier()
    pltpu.sync_copy(acc.at[pl.ds(base_n, _TN)], out.at[c, pl.ds(base_n, _TN)])


_agg_call = pl.kernel(
    _agg_body,
    out_type=jax.ShapeDtypeStruct((_NC, _NP, _H), jnp.float32),
    mesh=_mesh,
    scratch_types=[
        pltpu.VMEM((_EW,), jnp.int32),
        [pltpu.VMEM((_C,), jnp.int32)] * 3,
        [pltpu.VMEM((_C * 16 // 128, 128), jnp.float32)] * 3,
        [pltpu.VMEM((_C, _H), jnp.float32)] * 3,
        pltpu.VMEM_SHARED((_NP, _H), jnp.float32),
        [pltpu.SemaphoreType.DMA] * 3,
        [pltpu.SemaphoreType.DMA] * 3,
    ],
)


_NB = 512
_GRID = (_N + _NB - 1) // _NB


_EB = _E // _GRID


def _t0_body(feat, w0, b0, w1, ew, cnt, ew_col, g1, norms, ewb):
    inv_max = 1.0 / jnp.max(ew[...])
    deg_s = jnp.maximum(cnt[0, :, 0] + cnt[1, :, 0], 1.0)
    deg_d = jnp.maximum(cnt[0, :, 1] + cnt[1, :, 1], 1.0)
    no = lax.rsqrt(deg_s) * inv_max
    ni = lax.rsqrt(deg_d)
    h1 = jnp.dot(feat[...], w0[...], preferred_element_type=jnp.float32) + b0[...]
    g1[...] = jnp.dot(h1, w1[...], preferred_element_type=jnp.float32) * no[:, None]
    norms[...] = jnp.concatenate([no[:, None], ni[:, None]], axis=1)
    ewb[...] = jnp.broadcast_to(ew_col[...], (_EB, 16))


def _t0(feat, w0, b0, w1, ew2d, cnt, ew_col):
    return pl.pallas_call(
        _t0_body,
        grid=(_GRID,),
        in_specs=[
            pl.BlockSpec((_NB, 7), lambda i: (i, 0)),
            pl.BlockSpec((7, _H), lambda i: (0, 0)),
            pl.BlockSpec((1, _H), lambda i: (0, 0)),
            pl.BlockSpec((_H, _H), lambda i: (0, 0)),
            pl.BlockSpec((_E // 128, 128), lambda i: (0, 0)),
            pl.BlockSpec((_NC, _NB, _H), lambda i: (0, i, 0)),
            pl.BlockSpec((_EB, 1), lambda i: (i, 0)),
        ],
        out_specs=[
            pl.BlockSpec((_NB, _H), lambda i: (i, 0)),
            pl.BlockSpec((_NB, 2), lambda i: (i, 0)),
            pl.BlockSpec((_EB, 16), lambda i: (i, 0)),
        ],
        out_shape=[
            jax.ShapeDtypeStruct((_N, _H), jnp.float32),
            jax.ShapeDtypeStruct((_N, 2), jnp.float32),
            jax.ShapeDtypeStruct((_E, 16), jnp.float32),
        ],
    )(feat, w0, b0, w1, ew2d, cnt, ew_col)


def _t1_body(p, norms, b, w, g):
    ni = norms[:, 1]
    no = norms[:, 0]
    h = jnp.maximum((p[0] + p[1]) * ni[:, None] + b[...], 0.0)
    g[...] = jnp.dot(h, w[...], preferred_element_type=jnp.float32) * no[:, None]


def _t1(p, norms, b, w):
    return pl.pallas_call(
        _t1_body,
        grid=(_GRID,),
        in_specs=[
            pl.BlockSpec((_NC, _NB, _H), lambda i: (0, i, 0)),
            pl.BlockSpec((_NB, 2), lambda i: (i, 0)),
            pl.BlockSpec((1, _H), lambda i: (0, 0)),
            pl.BlockSpec((_H, _H), lambda i: (0, 0)),
        ],
        out_specs=pl.BlockSpec((_NB, _H), lambda i: (i, 0)),
        out_shape=jax.ShapeDtypeStruct((_N, _H), jnp.float32),
    )(p, norms, b, w)


def _t3_body(p, norms, b, cb, h_out, q_out, ind_out, loss):
    i = pl.program_id(0)
    ni = norms[:, 1]
    h = (p[0] + p[1]) * ni[:, None] + b[...]
    cbv = cb[...]
    d2 = (jnp.sum(h * h, axis=1, keepdims=True)
          - 2.0 * lax.dot_general(h, cbv, (((1,), (1,)), ((), ())),
                                  preferred_element_type=jnp.float32)
          + jnp.sum(cbv * cbv, axis=1)[None, :])
    m = jnp.min(d2, axis=1)
    iota = lax.broadcasted_iota(jnp.int32, (_NB, _K), 1)
    ind = jnp.min(jnp.where(d2 == m[:, None], iota, _K), axis=1)
    onehot = (iota == ind[:, None]).astype(jnp.float32)
    q = jnp.dot(onehot, cbv, preferred_element_type=jnp.float32)
    quant = h + (q - h)
    rid = i * _NB + lax.broadcasted_iota(jnp.int32, (_NB, 1), 0)
    valid = rid < _N
    sq = jnp.where(valid, (q - h) ** 2, 0.0)
    part = jnp.sum(sq) * (1.0 / (_N * _H))

    @pl.when(i == 0)
    def _():
        loss[...] = jnp.zeros_like(loss)

    loss[...] += part
    h_out[...] = h
    q_out[...] = quant
    ind_out[...] = ind[:, None]


def _t3(p, norms, b, cb):
    return pl.pallas_call(
        _t3_body,
        grid=(_GRID,),
        in_specs=[
            pl.BlockSpec((_NC, _NB, _H), lambda i: (0, i, 0)),
            pl.BlockSpec((_NB, 2), lambda i: (i, 0)),
            pl.BlockSpec((1, _H), lambda i: (0, 0)),
            pl.BlockSpec((_K, _H), lambda i: (0, 0)),
        ],
        out_specs=[
            pl.BlockSpec((_NB, _H), lambda i: (i, 0)),
            pl.BlockSpec((_NB, _H), lambda i: (i, 0)),
            pl.BlockSpec((_NB, 1), lambda i: (i, 0)),
            pl.BlockSpec((1, 1), lambda i: (0, 0)),
        ],
        out_shape=[
            jax.ShapeDtypeStruct((_N, _H), jnp.float32),
            jax.ShapeDtypeStruct((_N, _H), jnp.float32),
            jax.ShapeDtypeStruct((_N, 1), jnp.int32),
            jax.ShapeDtypeStruct((1, 1), jnp.float32),
        ],
    )(p, norms, b, cb)


def kernel(features, edge_index, edge_weight, epoch, W0, b0, W1, b1, W2, b2, W3, b3, codebook):
    src = edge_index[0]
    dst = edge_index[1]
    src3 = src.reshape(_NW, _NCHUNK, _C)
    dst3 = dst.reshape(_NW, _NCHUNK, _C)
    cnt = _deg_call(src3, dst3)
    ew2d = edge_weight.reshape(_E // 128, 128)
    g1, norms, ewb = _t0(features, W0, b0.reshape(1, _H), W1, ew2d, cnt,
                         edge_weight.reshape(_E, 1))
    ewb4 = ewb.reshape(_NW, _NCHUNK, _C * 16 // 128, 128)
    p1 = _agg_call(g1, src, dst3, ewb4)
    g2 = _t1(p1, norms, b1.reshape(1, _H), W2)
    p2 = _agg_call(g2, src, dst3, ewb4)
    g3 = _t1(p2, norms, b2.reshape(1, _H), W3)
    p3 = _agg_call(g3, src, dst3, ewb4)
    h, quant, ind, loss = _t3(p3, norms, b3.reshape(1, _H), codebook)
    return h, quant, jnp.reshape(loss, ()), ind.reshape(_N)
